# baseline (speedup 1.0000x reference)
import functools

import jax
import jax.numpy as jnp
from jax import lax
from jax.experimental import pallas as pl
from jax.experimental.pallas import tpu as pltpu
from jax.experimental.pallas import tpu_sc as plsc

N = 10000
F = 128
T = 4
NSINGLE = 5000
NP = 10240
BN = 640
NB = NP // BN
E = 320000
NTILES = 32
CHUNK = 128
CPT = 80
EP = NTILES * CPT * CHUNK
NBUF = 4
NGRP = CPT // NBUF
PAD_DST = N
SCALE = 4096.0


def _dot_t(a, w):
    return lax.dot_general(a, w, (((1,), (1,)), ((), ())),
                           preferred_element_type=jnp.float32)


def _quant(y):
    return jnp.clip(jnp.round(y * SCALE), -32768.0, 32767.0).astype(jnp.int16)


def _prep1_body(x_ref, w_ref, b_ref, bz_ref, tab_ref):
    x = x_ref[...]
    bz_ref[0] = _quant(_dot_t(x, w_ref[0]) + b_ref[...])
    bz_ref[1] = jnp.zeros((BN, F), jnp.int16)
    for t in range(T):
        tab_ref[t] = _quant(_dot_t(x, w_ref[1 + t]))


_prep1_call = pl.pallas_call(
    _prep1_body,
    grid=(NB,),
    in_specs=[
        pl.BlockSpec((BN, F), lambda j: (j, 0)),
        pl.BlockSpec((T + 1, F, F), lambda j: (0, 0, 0)),
        pl.BlockSpec((BN, F), lambda j: (j, 0)),
    ],
    out_specs=[
        pl.BlockSpec((2, BN, F), lambda j: (0, j, 0)),
        pl.BlockSpec((T, BN, F), lambda j: (0, j, 0)),
    ],
    out_shape=[
        jax.ShapeDtypeStruct((2, NP, F), jnp.int16),
        jax.ShapeDtypeStruct((T, NP, F), jnp.int16),
    ],
)


def _dequant_h(a_ref):
    a0 = a_ref[0].astype(jnp.float32)
    a1 = a_ref[1].astype(jnp.float32)
    return jnp.clip((a0 + a1) * (1.0 / SCALE), 0.0, 1.0)


def _prep2_body(a_ref, w_ref, b_ref, bz_ref, tab_ref):
    h = _dequant_h(a_ref)
    bz_ref[0] = _quant(_dot_t(h, w_ref[0]) + b_ref[...])
    bz_ref[1] = jnp.zeros((BN, F), jnp.int16)
    for t in range(T):
        tab_ref[t] = _quant(_dot_t(h, w_ref[1 + t]))


_prep2_call = pl.pallas_call(
    _prep2_body,
    grid=(NB,),
    in_specs=[
        pl.BlockSpec((2, BN, F), lambda j: (0, j, 0)),
        pl.BlockSpec((T + 1, F, F), lambda j: (0, 0, 0)),
        pl.BlockSpec((BN, F), lambda j: (j, 0)),
    ],
    out_specs=[
        pl.BlockSpec((2, BN, F), lambda j: (0, j, 0)),
        pl.BlockSpec((T, BN, F), lambda j: (0, j, 0)),
    ],
    out_shape=[
        jax.ShapeDtypeStruct((2, NP, F), jnp.int16),
        jax.ShapeDtypeStruct((T, NP, F), jnp.int16),
    ],
)


def _finish_body(a_ref, o_ref):
    o_ref[...] = _dequant_h(a_ref)


_finish_call = pl.pallas_call(
    _finish_body,
    grid=(NB,),
    in_specs=[pl.BlockSpec((2, BN, F), lambda j: (0, j, 0))],
    out_specs=pl.BlockSpec((BN, F), lambda j: (j, 0)),
    out_shape=jax.ShapeDtypeStruct((N, F), jnp.float32),
)


def _sc_body(table, bz, gidx, didx, out, gidx_v, didx_v, rows_v,
             acc, *sems):
    gsems = lambda b: sems[b]
    c = lax.axis_index("c")
    s = lax.axis_index("s")
    w = c * 16 + s

    pltpu.sync_copy(gidx.at[w], gidx_v)
    pltpu.sync_copy(didx.at[w], didx_v)

    r0 = s * BN
    pltpu.sync_copy(bz.at[c, pl.ds(r0, BN)], acc.at[pl.ds(r0, BN)])
    plsc.subcore_barrier()

    for b in range(NBUF):
        pltpu.async_copy(table.at[gidx_v.at[b]], rows_v.at[b], gsems(b))

    def grp(g, carry):
        for b in range(NBUF):
            j = g * NBUF + b
            pltpu.make_async_copy(
                table.at[gidx_v.at[j]], rows_v.at[b], gsems(b)).wait()
            pltpu.sync_copy(rows_v.at[b], acc.at[didx_v.at[j]], add=True)

            @pl.when(g < NGRP - 1)
            def _refire():
                pltpu.async_copy(
                    table.at[gidx_v.at[j + NBUF]], rows_v.at[b], gsems(b))
        return carry

    lax.fori_loop(0, NGRP, grp, 0)

    plsc.subcore_barrier()
    pltpu.sync_copy(acc.at[pl.ds(r0, BN)], out.at[c, pl.ds(r0, BN)])


@functools.cache
def _get_sc_call():
    return functools.partial(
        pl.kernel,
        out_type=jax.ShapeDtypeStruct((2, NP, F), jnp.int16),
        mesh=plsc.VectorSubcoreMesh(
            core_axis_name="c", subcore_axis_name="s"),
        compiler_params=pltpu.CompilerParams(use_tc_tiling_on_sc=False),
        scratch_types=[
            pltpu.VMEM((CPT, CHUNK), jnp.int32),
            pltpu.VMEM((CPT, CHUNK), jnp.int32),
            pltpu.VMEM((NBUF, CHUNK, F), jnp.int16),
            pltpu.VMEM_SHARED((NP, F), jnp.int16),
        ] + [pltpu.SemaphoreType.DMA] * NBUF,
    )(_sc_body)


def _bias_rows(bs, bp):
    sel = (jnp.arange(NP) < NSINGLE)[:, None]
    return jnp.where(sel, bs[None, :], bp[None, :])


def kernel(x, edge_index, edge_type, A1, B1, bs1, bp1, A2, B2, bs2, bp2):
    src = edge_index[0].astype(jnp.int32)
    dst = edge_index[1].astype(jnp.int32)
    et = edge_type.astype(jnp.int32)

    gidx = jnp.concatenate([et * NP + src, jnp.zeros((EP - E,), jnp.int32)])
    didx = jnp.concatenate([dst, jnp.full((EP - E,), PAD_DST, jnp.int32)])
    gidx = gidx.reshape(NTILES, CPT, CHUNK)
    didx = didx.reshape(NTILES, CPT, CHUNK)

    xp = jnp.pad(x, ((0, NP - N), (0, 0)))
    bias1 = _bias_rows(bs1, bp1)
    bias2 = _bias_rows(bs2, bp2)

    sc_call = _get_sc_call()
    w1 = jnp.concatenate([A1[None], B1], axis=0)
    bz1, table1 = _prep1_call(xp, w1, bias1)
    acc1 = sc_call(table1.reshape(T * NP, F), bz1, gidx, didx)

    w2 = jnp.concatenate([A2[None], B2], axis=0)
    bz2, table2 = _prep2_call(acc1, w2, bias2)
    acc2 = sc_call(table2.reshape(T * NP, F), bz2, gidx, didx)

    return _finish_call(acc2)
